# baseline (speedup 1.0000x reference)
import functools
import math

import jax
import jax.numpy as jnp
from jax import lax
from jax.experimental import pallas as pl
from jax.experimental.pallas import tpu as pltpu
from jax.experimental.pallas import tpu_sc as plsc

D_MODEL = 64
SCALE = math.sqrt(D_MODEL)
NUM_WORKERS = 32
X_ROWS = 4096
X_COLS = 200
XR_PER_WORKER = X_ROWS // NUM_WORKERS
NBUF = 4
LANES = 16


def _make_kernel():
    mesh = plsc.VectorSubcoreMesh(core_axis_name="c", subcore_axis_name="s")

    @functools.partial(
        pl.kernel,
        out_type=jax.ShapeDtypeStruct((X_ROWS, X_COLS, D_MODEL), jnp.float32),
        mesh=mesh,
        compiler_params=pltpu.CompilerParams(use_tc_tiling_on_sc=False),
        scratch_types=(
            [pltpu.VMEM((XR_PER_WORKER, X_COLS), jnp.int32)]
            + [pltpu.VMEM((X_COLS, D_MODEL), jnp.float32)] * NBUF
            + [pltpu.SemaphoreType.DMA] * (2 * NBUF)
        ),
    )
    def gather_scale(idx_hbm, table_hbm, out_hbm, idx_all, *bufs_and_sems):
        rows = list(bufs_and_sems[:NBUF])
        gsem = list(bufs_and_sems[NBUF:2 * NBUF])
        osem = list(bufs_and_sems[2 * NBUF:])
        wid = lax.axis_index("s") * 2 + lax.axis_index("c")
        base = wid * XR_PER_WORKER

        pltpu.sync_copy(idx_hbm.at[pl.ds(base, XR_PER_WORKER)], idx_all)

        def gather_desc(c, b):
            src = table_hbm.at[idx_all.at[c]]
            return pltpu.make_async_copy(src, rows[b], gsem[b])

        def out_desc(c, b):
            return pltpu.make_async_copy(rows[b], out_hbm.at[base + c], osem[b])

        for c0 in range(NBUF - 1):
            gather_desc(c0, c0).start()

        def ring_body(p, carry):
            for b in range(NBUF):
                c = p * NBUF + b
                bprev = (b - 1) % NBUF
                gather_desc(c, b).wait()

                @pl.when(c >= 1)
                def _wait_prev_out():
                    out_desc(c - 1, bprev).wait()

                @pl.when(c + NBUF - 1 < XR_PER_WORKER)
                def _start_next_gather():
                    gather_desc(c + NBUF - 1, bprev).start()

                def scale_body(t, carry2):
                    for j in range(D_MODEL // LANES):
                        sl = pl.ds(j * LANES, LANES)
                        rows[b][t, sl] = rows[b][t, sl] * SCALE
                    return carry2

                lax.fori_loop(0, X_COLS, scale_body, 0, unroll=4)
                out_desc(c, b).start()
            return carry

        lax.fori_loop(0, XR_PER_WORKER // NBUF, ring_body, 0)
        out_desc(XR_PER_WORKER - 1, (XR_PER_WORKER - 1) % NBUF).wait()

    return gather_scale


_gather_scale = _make_kernel()


def kernel(x, embed):
    return _gather_scale(x, embed)
